# baseline (speedup 1.0000x reference)
import numpy as np
import jax
import jax.numpy as jnp
from jax import lax
from jax.experimental import pallas as pl
from jax.experimental.pallas import tpu as pltpu
from jax.experimental.pallas import tpu_sc as plsc

VOCAB = 100000
EMB = 128
HID = 64
OUT = 256
B = 4096
L = 50

NC = 2
NS = 16
NW = NC * NS
NH = 2
B2 = B // NH
BPW = B2 // NW


def _pool_body(tok_hbm, table_hbm, zero_hbm, out_hbm, tok_v, acc, sem):
    c = lax.axis_index("c")
    s = lax.axis_index("s")
    wid = c * NS + s

    pltpu.sync_copy(tok_hbm.at[wid], tok_v)
    pltpu.sync_copy(zero_hbm, acc)

    def fire(t, carry):
        pltpu.async_copy(table_hbm.at[tok_v.at[t]], acc, sem, add=True)
        return carry

    lax.fori_loop(0, L, fire, 0)

    def drain(t, carry):
        pltpu.make_async_copy(table_hbm.at[tok_v.at[0]], acc, sem).wait()
        return carry

    lax.fori_loop(0, L, drain, 0)

    pltpu.sync_copy(acc, out_hbm.at[pl.ds(wid * BPW, BPW)])


_pool = pl.kernel(
    _pool_body,
    mesh=plsc.VectorSubcoreMesh(core_axis_name="c", subcore_axis_name="s"),
    out_type=jax.ShapeDtypeStruct((B2, EMB), jnp.float32),
    scratch_types=[
        pltpu.VMEM((L, BPW), jnp.int32),
        pltpu.VMEM((BPW, EMB), jnp.float32),
        pltpu.SemaphoreType.DMA,
    ],
)

MB = 512


def _mlp_body(x_ref, w1_ref, b1_ref, w2_ref, b2_ref, w3_ref, b3_ref, o_ref):
    x = x_ref[...] * jnp.float32(1.0 / L)
    h = jnp.dot(x, w1_ref[...], preferred_element_type=jnp.float32)
    h = jnp.maximum(h + b1_ref[...], 0.0)
    h = jnp.dot(h, w2_ref[...], preferred_element_type=jnp.float32)
    h = jnp.maximum(h + b2_ref[...], 0.0)
    o = jnp.dot(h, w3_ref[...], preferred_element_type=jnp.float32)
    o_ref[...] = o + b3_ref[...]


_mlp = pl.pallas_call(
    _mlp_body,
    grid=(B2 // MB,),
    in_specs=[
        pl.BlockSpec((MB, EMB), lambda i: (i, 0)),
        pl.BlockSpec((EMB, HID), lambda i: (0, 0)),
        pl.BlockSpec((1, HID), lambda i: (0, 0)),
        pl.BlockSpec((HID, HID), lambda i: (0, 0)),
        pl.BlockSpec((1, HID), lambda i: (0, 0)),
        pl.BlockSpec((HID, OUT), lambda i: (0, 0)),
        pl.BlockSpec((1, OUT), lambda i: (0, 0)),
    ],
    out_specs=pl.BlockSpec((MB, OUT), lambda i: (i, 0)),
    out_shape=jax.ShapeDtypeStruct((B2, OUT), jnp.float32),
)


def kernel(token_ids, emb_table, W1, b1, W2, b2, W3, b3):
    tok = token_ids.astype(jnp.int32).reshape(NH, NW, BPW, L)
    tok = tok.transpose(0, 1, 3, 2)
    zero = jnp.zeros((BPW, EMB), jnp.float32)
    b1r, b2r, b3r = b1.reshape(1, HID), b2.reshape(1, HID), b3.reshape(1, OUT)
    outs = []
    sums = [_pool(tok[h], emb_table, zero) for h in range(NH)]
    for h in range(NH):
        outs.append(_mlp(sums[h], W1, b1r, W2, b2r, W3, b3r))
    return jnp.concatenate(outs, axis=0)
